# Initial kernel scaffold; baseline (speedup 1.0000x reference)
import jax
import jax.numpy as jnp
from jax.experimental import pallas as pl


def kernel(x, table):
    raise NotImplementedError("write your pallas kernel here")


# speedup vs baseline: 1.0282x; 1.0282x over previous
import math

import numpy as np
import jax
import jax.numpy as jnp
from jax import lax
from jax.experimental import pallas as pl
from jax.experimental.pallas import tpu as pltpu
from jax.experimental.pallas import tpu_sc as plsc

D_MODEL = 128
VOCAB = 1000000
B_TOTAL = 4096 * 200
NC, NS, L = 2, 16, 16
NW = NC * NS
B_PER_W = B_TOTAL // NW
CHUNK = 128
NCH = B_PER_W // CHUNK
NBUF = 4
AHEAD = 2
NOUT = NCH // NBUF
SCALE = np.float32(math.sqrt(D_MODEL))

_mesh = plsc.VectorSubcoreMesh(core_axis_name="c", subcore_axis_name="s")


def _body(x_ref, table_ref, out_ref, idx_v, rows, sem):
    wid = lax.axis_index("s") * NC + lax.axis_index("c")
    base = wid * B_PER_W

    pltpu.sync_copy(x_ref.at[wid], idx_v)

    def chunk_step(g, carry):
        pltpu.async_copy(table_ref.at[idx_v.at[g]], rows, sem).wait()

        def row_body(r, c):
            for k in range(D_MODEL // L):
                sl = pl.ds(k * L, L)
                rows[r, sl] = rows[r, sl] * SCALE
            return c
        lax.fori_loop(0, CHUNK, row_body, 0)

        pltpu.sync_copy(rows, out_ref.at[pl.ds(base + g * CHUNK, CHUNK)])
        return carry

    lax.fori_loop(0, NCH, chunk_step, 0)


@jax.jit
def _run(x32, table):
    def body(x_ref, table_ref, out_ref, idx_v, rows, sem):
        _body(x_ref, table_ref, out_ref, idx_v, rows, sem)

    k = pl.kernel(
        body,
        out_type=jax.ShapeDtypeStruct((B_TOTAL, D_MODEL), jnp.float32),
        mesh=_mesh,
        scratch_types=(
            pltpu.VMEM((NCH, CHUNK), jnp.int32),
            pltpu.VMEM((CHUNK, D_MODEL), jnp.float32),
            pltpu.SemaphoreType.DMA,
        ),
    )
    return k(x32, table)


def kernel(x, table):
    x32 = x.astype(jnp.int32).reshape(NW, NCH, CHUNK)
    out = _run(x32, table)
    return out.reshape(x.shape[0], x.shape[1], D_MODEL)


# speedup vs baseline: 1.6205x; 1.5760x over previous
import math

import numpy as np
import jax
import jax.numpy as jnp
from jax import lax
from jax.experimental import pallas as pl
from jax.experimental.pallas import tpu as pltpu
from jax.experimental.pallas import tpu_sc as plsc

D_MODEL = 128
VOCAB = 1000000
B_TOTAL = 4096 * 200
NC, NS, L = 2, 16, 16
NW = NC * NS
B_PER_W = B_TOTAL // NW
CHUNK = 128
NCH = B_PER_W // CHUNK
NBUF = 4
AHEAD = 2
NOUT = NCH // NBUF
SCALE = np.float32(math.sqrt(D_MODEL))

_mesh = plsc.VectorSubcoreMesh(core_axis_name="c", subcore_axis_name="s")


def _body(x_ref, table_ref, out_ref, idx_v, rows, gsem):
    wid = lax.axis_index("s") * NC + lax.axis_index("c")
    base = wid * B_PER_W

    pltpu.sync_copy(x_ref.at[wid], idx_v)

    def gather_start(g, b):
        pltpu.async_copy(table_ref.at[idx_v.at[g]], rows[b], gsem[b])

    def gather_wait(g, b):
        pltpu.make_async_copy(table_ref.at[idx_v.at[g]], rows[b],
                              gsem[b]).wait()

    def scale(b):
        def row_body(r, c):
            for k in range(D_MODEL // L):
                sl = pl.ds(k * L, L)
                rows[b][r, sl] = rows[b][r, sl] * SCALE
            return c
        lax.fori_loop(0, CHUNK, row_body, 0)

    def chunk_step(g, b, start_next):
        gather_wait(g, b)
        if start_next:
            gather_start(g + 1, 1 - b)
        scale(b)
        pltpu.sync_copy(rows[b], out_ref.at[pl.ds(base + g * CHUNK, CHUNK)])

    gather_start(0, 0)

    def trip(i, carry):
        g0 = i * 2
        for b in range(2):
            chunk_step(g0 + b, b, start_next=True)
        return carry
    lax.fori_loop(0, NCH // 2 - 1, trip, 0)

    chunk_step(NCH - 2, 0, start_next=True)
    chunk_step(NCH - 1, 1, start_next=False)


@jax.jit
def _run(x32, table):
    def body(x_ref, table_ref, out_ref, idx_v, r0, r1, g0, g1):
        _body(x_ref, table_ref, out_ref, idx_v, (r0, r1), (g0, g1))

    k = pl.kernel(
        body,
        out_type=jax.ShapeDtypeStruct((B_TOTAL, D_MODEL), jnp.float32),
        mesh=_mesh,
        scratch_types=(
            pltpu.VMEM((NCH, CHUNK), jnp.int32),
            pltpu.VMEM((CHUNK, D_MODEL), jnp.float32),
            pltpu.VMEM((CHUNK, D_MODEL), jnp.float32),
            pltpu.SemaphoreType.DMA,
            pltpu.SemaphoreType.DMA,
        ),
    )
    return k(x32, table)


def kernel(x, table):
    x32 = x.astype(jnp.int32).reshape(NW, NCH, CHUNK)
    out = _run(x32, table)
    return out.reshape(x.shape[0], x.shape[1], D_MODEL)


# speedup vs baseline: 1.8732x; 1.1559x over previous
import math

import numpy as np
import jax
import jax.numpy as jnp
from jax import lax
from jax.experimental import pallas as pl
from jax.experimental.pallas import tpu as pltpu
from jax.experimental.pallas import tpu_sc as plsc

D_MODEL = 128
VOCAB = 1000000
B_TOTAL = 4096 * 200
NC, NS, L = 2, 16, 16
NW = NC * NS
B_PER_W = B_TOTAL // NW
CHUNK = 128
NCH = B_PER_W // CHUNK
NBUF = 4
AHEAD = 2
NOUT = NCH // NBUF
SCALE = np.float32(math.sqrt(D_MODEL))

_mesh = plsc.VectorSubcoreMesh(core_axis_name="c", subcore_axis_name="s")


def _body(x_ref, table_ref, out_ref, idx_v, rows, gsem, ssem):
    wid = lax.axis_index("s") * NC + lax.axis_index("c")
    base = wid * B_PER_W

    pltpu.sync_copy(x_ref.at[wid], idx_v)

    def gather_start(g, b):
        pltpu.async_copy(table_ref.at[idx_v.at[g]], rows[b], gsem[b])

    def gather_wait(g, b):
        pltpu.make_async_copy(table_ref.at[idx_v.at[g]], rows[b],
                              gsem[b]).wait()

    def scatter_start(g, b):
        pltpu.async_copy(rows[b], out_ref.at[pl.ds(base + g * CHUNK, CHUNK)],
                         ssem[b])

    def scatter_wait(g, b):
        pltpu.make_async_copy(rows[b],
                              out_ref.at[pl.ds(base + g * CHUNK, CHUNK)],
                              ssem[b]).wait()

    def scale(b):
        def row_body(r, c):
            for k in range(D_MODEL // L):
                sl = pl.ds(k * L, L)
                rows[b][r, sl] = rows[b][r, sl] * SCALE
            return c
        lax.fori_loop(0, CHUNK, row_body, 0)

    def chunk_step(g, b, do_swait, do_gstart):
        gather_wait(g, b)
        scale(b)
        scatter_start(g, b)
        bn = (b + AHEAD) % NBUF
        if do_swait:
            scatter_wait(g - (NBUF - AHEAD), bn)
        if do_gstart:
            gather_start(g + AHEAD, bn)

    for g in range(AHEAD):
        gather_start(g, g % NBUF)

    for b in range(NBUF):
        chunk_step(b, b, do_swait=(b >= NBUF - AHEAD), do_gstart=True)

    def trip(i, carry):
        g0 = i * NBUF
        for b in range(NBUF):
            chunk_step(g0 + b, b, do_swait=True, do_gstart=True)
        return carry
    lax.fori_loop(1, NOUT - 1, trip, 0)

    g0 = NCH - NBUF
    for b in range(NBUF):
        live = b + AHEAD < NBUF
        chunk_step(g0 + b, b, do_swait=live, do_gstart=live)

    for b in range(NBUF):
        scatter_wait(NCH - NBUF + b, b)


@jax.jit
def _run(x32, table):
    def body(x_ref, table_ref, out_ref, idx_v, r0, r1, r2, r3,
             g0, g1, g2, g3, s0, s1, s2, s3):
        _body(x_ref, table_ref, out_ref, idx_v,
              (r0, r1, r2, r3), (g0, g1, g2, g3), (s0, s1, s2, s3))

    k = pl.kernel(
        body,
        out_type=jax.ShapeDtypeStruct((B_TOTAL, D_MODEL), jnp.float32),
        mesh=_mesh,
        scratch_types=(
            [pltpu.VMEM((NCH, CHUNK), jnp.int32)]
            + [pltpu.VMEM((CHUNK, D_MODEL), jnp.float32)] * NBUF
            + [pltpu.SemaphoreType.DMA] * (2 * NBUF)
        ),
    )
    return k(x32, table)


def kernel(x, table):
    x32 = x.astype(jnp.int32).reshape(NW, NCH, CHUNK)
    out = _run(x32, table)
    return out.reshape(x.shape[0], x.shape[1], D_MODEL)
